# baseline (speedup 1.0000x reference)
import functools

import jax
import jax.numpy as jnp
import numpy as np
from jax.experimental import pallas as pl
from jax.experimental.pallas import tpu as pltpu

_INT_MIN = -2147483648


def _sort_key(x):
    bits = jax.lax.bitcast_convert_type(x, jnp.int32)
    return jnp.where(bits < 0, bits ^ np.int32(0x7FFFFFFF), bits)


def _bdot(a, b):
    return jnp.dot(a.astype(jnp.bfloat16), b.astype(jnp.bfloat16),
                   preferred_element_type=jnp.float32)


def _fused_kernel(a_ref, tail_ref, x_ref, w1_ref, b1_ref, wm_ref, bm_ref,
                  g_ref, out_ref, y_ref, pcol_ref, cmask_ref, thr_ref,
                  *, k, n, tr, n_tiles, c_tiles):
    i = pl.program_id(0)
    u_tiles = n_tiles - c_tiles

    @pl.when(i == 0)
    def _():
        y_ref[...] = _bdot(x_ref[...], w1_ref[...]).astype(jnp.bfloat16)

    @pl.when(i < n_tiles)
    def _logits_step():
        r = jnp.where(i < c_tiles, u_tiles + i, i - c_tiles)
        t_off = jnp.where(i < c_tiles, i, 0) * tr
        a_tail = tail_ref[pl.ds(pl.multiple_of(t_off, tr), tr), :]
        a = jnp.where(i < c_tiles, a_tail, a_ref[...])
        h = jnp.maximum(
            jnp.dot(a.astype(jnp.bfloat16), y_ref[...],
                    preferred_element_type=jnp.float32) + b1_ref[...], 0.0)
        p = _bdot(h, wm_ref[...]) + bm_ref[...] + g_ref[...]
        pcol_ref[pl.ds(pl.multiple_of(r * tr, tr), tr), :] = p

    @pl.when(i == n_tiles)
    def _select():
        keys = _sort_key(jnp.transpose(pcol_ref[...], (1, 0)))
        idx = jax.lax.broadcasted_iota(jnp.int32, (1, n), 1)

        def val_body(t, prefix_u):
            cand_u = prefix_u | jax.lax.shift_left(np.int32(1),
                                                   np.int32(31) - t)
            cand_s = cand_u ^ _INT_MIN
            cnt = jnp.sum((keys >= cand_s).astype(jnp.int32))
            return jnp.where(cnt >= k, cand_u, prefix_u)

        prefix_u = jax.lax.fori_loop(0, 32, val_body, np.int32(0))
        thr = prefix_u ^ _INT_MIN

        need = k - jnp.sum((keys > thr).astype(jnp.int32))
        eq = keys == thr

        def idx_body(t, prefix):
            b = np.int32(11) - t
            cap = prefix | (jax.lax.shift_left(np.int32(1), b) - 1)
            cnt = jnp.sum((eq & (idx <= cap)).astype(jnp.int32))
            return jnp.where(cnt >= need, prefix,
                             prefix | jax.lax.shift_left(np.int32(1), b))

        tidx = jax.lax.fori_loop(0, 12, idx_body, np.int32(0))
        thr_ref[0] = thr
        thr_ref[1] = tidx
        cmask_ref[...] = ((keys > thr) |
                          ((keys == thr) & (idx <= tidx))).astype(jnp.float32)

    @pl.when(i >= n_tiles)
    def _mask_step():
        j = i - n_tiles
        thr = thr_ref[0]
        tidx = thr_ref[1]
        rk = _sort_key(pcol_ref[pl.ds(pl.multiple_of(j * tr, tr), tr), :])
        ridx = jax.lax.broadcasted_iota(jnp.int32, (tr, 1), 0) + j * tr
        rm = ((rk > thr) |
              ((rk == thr) & (ridx <= tidx))).astype(jnp.float32)
        t_off = jnp.where(j >= u_tiles, j - u_tiles, 0) * tr
        a_tail = tail_ref[pl.ds(pl.multiple_of(t_off, tr), tr), :]
        a = jnp.where(j >= u_tiles, a_tail, a_ref[...])
        out_ref[...] = a * rm * cmask_ref[...]


def kernel(A, X, W1, b1, Wm, bm):
    n, d = X.shape
    h = W1.shape[1]
    k = max(1, int(0.5 * n))

    u = jax.random.uniform(jax.random.key(42), (n, 1), dtype=jnp.float32)
    g = -jnp.log(-jnp.log(u + 1e-08) + 1e-08)

    tr = 128
    n_tiles = n // tr
    c_tiles = 16
    u_tiles = n_tiles - c_tiles

    def a_map(i):
        return (jnp.where(i < n_tiles,
                          jnp.maximum(i - c_tiles, 0),
                          jnp.minimum(i - n_tiles, u_tiles - 1)), 0)

    def g_map(i):
        return (jnp.where(i < c_tiles, u_tiles + i,
                          jnp.clip(i - c_tiles, 0, n_tiles - 1)), 0)

    body = functools.partial(_fused_kernel, k=k, n=n, tr=tr,
                             n_tiles=n_tiles, c_tiles=c_tiles)
    A_aug = pl.pallas_call(
        body,
        grid=(2 * n_tiles,),
        in_specs=[
            pl.BlockSpec((tr, n), a_map),
            pl.BlockSpec((c_tiles * tr, n),
                         lambda i: (u_tiles // c_tiles, 0)),
            pl.BlockSpec((n, d), lambda i: (0, 0)),
            pl.BlockSpec((d, h), lambda i: (0, 0)),
            pl.BlockSpec((1, h), lambda i: (0, 0)),
            pl.BlockSpec((h, 1), lambda i: (0, 0)),
            pl.BlockSpec((1, 1), lambda i: (0, 0)),
            pl.BlockSpec((tr, 1), g_map),
        ],
        out_specs=pl.BlockSpec((tr, n),
                               lambda i: (jnp.maximum(i - n_tiles, 0), 0)),
        out_shape=jax.ShapeDtypeStruct((n, n), jnp.float32),
        scratch_shapes=[pltpu.VMEM((n, h), jnp.bfloat16),
                        pltpu.VMEM((n, 1), jnp.float32),
                        pltpu.VMEM((1, n), jnp.float32),
                        pltpu.SMEM((2,), jnp.int32)],
        compiler_params=pltpu.CompilerParams(
            vmem_limit_bytes=56 * 1024 * 1024,
            dimension_semantics=("arbitrary",),
        ),
    )(A, A, X, W1, b1.reshape(1, h), Wm, bm.reshape(1, 1), g)
    return A_aug


# speedup vs baseline: 1.1393x; 1.1393x over previous
import functools

import jax
import jax.numpy as jnp
import numpy as np
from jax.experimental import pallas as pl
from jax.experimental.pallas import tpu as pltpu

_INT_MIN = -2147483648


def _sort_key(x):
    bits = jax.lax.bitcast_convert_type(x, jnp.int32)
    return jnp.where(bits < 0, bits ^ np.int32(0x7FFFFFFF), bits)


def _bdot(a, b):
    return jnp.dot(a.astype(jnp.bfloat16), b.astype(jnp.bfloat16),
                   preferred_element_type=jnp.float32)


def _y_kernel(x_ref, w1_ref, y_ref):
    y_ref[...] = _bdot(x_ref[...], w1_ref[...]).astype(jnp.bfloat16)


def _fused_kernel(a_ref, tail_ref, y_ref, b1_ref, wm_ref, bm_ref,
                  g_ref, out_ref, pcol_ref, cmask_ref, thr_ref,
                  *, k, n, tr, n_tiles, c_tiles):
    i = pl.program_id(0)
    u_tiles = n_tiles - c_tiles

    @pl.when(i < n_tiles)
    def _logits_step():
        r = jnp.where(i < c_tiles, u_tiles + i, i - c_tiles)
        t_off = jnp.where(i < c_tiles, i, 0) * tr
        a_tail = tail_ref[pl.ds(pl.multiple_of(t_off, tr), tr), :]
        a = jnp.where(i < c_tiles, a_tail, a_ref[...])
        h = jnp.maximum(
            jnp.dot(a.astype(jnp.bfloat16), y_ref[...],
                    preferred_element_type=jnp.float32) + b1_ref[...], 0.0)
        p = _bdot(h, wm_ref[...]) + bm_ref[...] + g_ref[...]
        pcol_ref[pl.ds(pl.multiple_of(r * tr, tr), tr), :] = p

    @pl.when(i == n_tiles)
    def _select():
        keys = _sort_key(jnp.transpose(pcol_ref[...], (1, 0)))
        idx = jax.lax.broadcasted_iota(jnp.int32, (1, n), 1)

        def val_body(t, prefix_u):
            cand_u = prefix_u | jax.lax.shift_left(np.int32(1),
                                                   np.int32(31) - t)
            cand_s = cand_u ^ _INT_MIN
            cnt = jnp.sum((keys >= cand_s).astype(jnp.int32))
            return jnp.where(cnt >= k, cand_u, prefix_u)

        prefix_u = jax.lax.fori_loop(0, 32, val_body, np.int32(0))
        thr = prefix_u ^ _INT_MIN

        need = k - jnp.sum((keys > thr).astype(jnp.int32))
        eq = keys == thr

        def idx_body(t, prefix):
            b = np.int32(11) - t
            cap = prefix | (jax.lax.shift_left(np.int32(1), b) - 1)
            cnt = jnp.sum((eq & (idx <= cap)).astype(jnp.int32))
            return jnp.where(cnt >= need, prefix,
                             prefix | jax.lax.shift_left(np.int32(1), b))

        tidx = jax.lax.fori_loop(0, 12, idx_body, np.int32(0))
        thr_ref[0] = thr
        thr_ref[1] = tidx
        cmask_ref[...] = ((keys > thr) |
                          ((keys == thr) & (idx <= tidx))).astype(jnp.float32)

    @pl.when(i >= n_tiles)
    def _mask_step():
        j = i - n_tiles
        thr = thr_ref[0]
        tidx = thr_ref[1]
        rk = _sort_key(pcol_ref[pl.ds(pl.multiple_of(j * tr, tr), tr), :])
        ridx = jax.lax.broadcasted_iota(jnp.int32, (tr, 1), 0) + j * tr
        rm = ((rk > thr) |
              ((rk == thr) & (ridx <= tidx))).astype(jnp.float32)
        t_off = jnp.where(j >= u_tiles, j - u_tiles, 0) * tr
        a_tail = tail_ref[pl.ds(pl.multiple_of(t_off, tr), tr), :]
        a = jnp.where(j >= u_tiles, a_tail, a_ref[...])
        out_ref[...] = a * rm * cmask_ref[...]


def kernel(A, X, W1, b1, Wm, bm):
    n, d = X.shape
    h = W1.shape[1]
    k = max(1, int(0.5 * n))

    u = jax.random.uniform(jax.random.key(42), (n, 1), dtype=jnp.float32)
    g = -jnp.log(-jnp.log(u + 1e-08) + 1e-08)

    tr = 256
    n_tiles = n // tr
    c_tiles = 8
    u_tiles = n_tiles - c_tiles

    def a_map(i):
        return (jnp.where(i < n_tiles,
                          jnp.maximum(i - c_tiles, 0),
                          jnp.minimum(i - n_tiles, u_tiles - 1)), 0)

    def g_map(i):
        return (jnp.where(i < c_tiles, u_tiles + i,
                          jnp.clip(i - c_tiles, 0, n_tiles - 1)), 0)

    Y = pl.pallas_call(
        _y_kernel,
        out_shape=jax.ShapeDtypeStruct((n, h), jnp.bfloat16),
    )(X, W1)

    body = functools.partial(_fused_kernel, k=k, n=n, tr=tr,
                             n_tiles=n_tiles, c_tiles=c_tiles)
    A_aug = pl.pallas_call(
        body,
        grid=(2 * n_tiles,),
        in_specs=[
            pl.BlockSpec((tr, n), a_map),
            pl.BlockSpec((c_tiles * tr, n),
                         lambda i: (u_tiles // c_tiles, 0)),
            pl.BlockSpec((n, h), lambda i: (0, 0)),
            pl.BlockSpec((1, h), lambda i: (0, 0)),
            pl.BlockSpec((h, 1), lambda i: (0, 0)),
            pl.BlockSpec((1, 1), lambda i: (0, 0)),
            pl.BlockSpec((tr, 1), g_map),
        ],
        out_specs=pl.BlockSpec((tr, n),
                               lambda i: (jnp.maximum(i - n_tiles, 0), 0)),
        out_shape=jax.ShapeDtypeStruct((n, n), jnp.float32),
        scratch_shapes=[pltpu.VMEM((n, 1), jnp.float32),
                        pltpu.VMEM((1, n), jnp.float32),
                        pltpu.SMEM((2,), jnp.int32)],
        compiler_params=pltpu.CompilerParams(
            vmem_limit_bytes=60_000_000,
            dimension_semantics=("arbitrary",),
        ),
    )(A, A, Y, b1.reshape(1, h), Wm, bm.reshape(1, 1), g)
    return A_aug


# speedup vs baseline: 1.1539x; 1.0128x over previous
import functools

import jax
import jax.numpy as jnp
import numpy as np
from jax.experimental import pallas as pl
from jax.experimental.pallas import tpu as pltpu

_INT_MIN = -2147483648


def _sort_key(x):
    bits = jax.lax.bitcast_convert_type(x, jnp.int32)
    return jnp.where(bits < 0, bits ^ np.int32(0x7FFFFFFF), bits)


def _bdot(a, b):
    return jnp.dot(a.astype(jnp.bfloat16), b.astype(jnp.bfloat16),
                   preferred_element_type=jnp.float32)


def _y_kernel(x_ref, w1_ref, y_ref):
    y_ref[...] = _bdot(x_ref[...], w1_ref[...]).astype(jnp.bfloat16)


def _fused_kernel(a_ref, tail_ref, y_ref, b1_ref, wm_ref, bm_ref,
                  g_ref, out_ref, pcol_ref, cmask_ref, thr_ref,
                  *, k, n, tr, n_tiles, c_tiles):
    i = pl.program_id(0)
    u_tiles = n_tiles - c_tiles

    @pl.when(i < n_tiles)
    def _logits_step():
        r = jnp.where(i < c_tiles, u_tiles + i, i - c_tiles)
        t_off = jnp.where(i < c_tiles, i, 0) * tr
        a_tail = tail_ref[pl.ds(pl.multiple_of(t_off, tr), tr), :]
        a = jnp.where(i < c_tiles, a_tail, a_ref[...])
        h = jnp.maximum(
            jnp.dot(a.astype(jnp.bfloat16), y_ref[...],
                    preferred_element_type=jnp.float32) + b1_ref[...], 0.0)
        p = _bdot(h, wm_ref[...]) + bm_ref[...] + g_ref[...]
        pcol_ref[pl.ds(pl.multiple_of(r * tr, tr), tr), :] = p

    @pl.when(i == n_tiles)
    def _select():
        keys = _sort_key(jnp.transpose(pcol_ref[...], (1, 0)))
        idx = jax.lax.broadcasted_iota(jnp.int32, (1, n), 1)

        def val_body(t, prefix_u):
            cand_u = prefix_u | jax.lax.shift_left(np.int32(1),
                                                   np.int32(31) - t)
            cand_s = cand_u ^ _INT_MIN
            cnt = jnp.sum((keys >= cand_s).astype(jnp.int32))
            return jnp.where(cnt >= k, cand_u, prefix_u)

        prefix_u = jax.lax.fori_loop(0, 32, val_body, np.int32(0))
        thr = prefix_u ^ _INT_MIN

        need = k - jnp.sum((keys > thr).astype(jnp.int32))
        eq = keys == thr

        def idx_body(t, prefix):
            b = np.int32(11) - t
            cap = prefix | (jax.lax.shift_left(np.int32(1), b) - 1)
            cnt = jnp.sum((eq & (idx <= cap)).astype(jnp.int32))
            return jnp.where(cnt >= need, prefix,
                             prefix | jax.lax.shift_left(np.int32(1), b))

        tidx = jax.lax.fori_loop(0, 12, idx_body, np.int32(0))
        thr_ref[0] = thr
        thr_ref[1] = tidx
        cmask_ref[...] = ((keys > thr) |
                          ((keys == thr) & (idx <= tidx))).astype(jnp.float32)

    @pl.when(i >= n_tiles)
    def _mask_step():
        j = i - n_tiles
        thr = thr_ref[0]
        tidx = thr_ref[1]
        rk = _sort_key(pcol_ref[pl.ds(pl.multiple_of(j * tr, tr), tr), :])
        ridx = jax.lax.broadcasted_iota(jnp.int32, (tr, 1), 0) + j * tr
        rm = ((rk > thr) |
              ((rk == thr) & (ridx <= tidx))).astype(jnp.float32)
        t_off = jnp.where(j >= u_tiles, j - u_tiles, 0) * tr
        a_tail = tail_ref[pl.ds(pl.multiple_of(t_off, tr), tr), :]
        a = jnp.where(j >= u_tiles, a_tail, a_ref[...])
        out_ref[...] = a * rm * cmask_ref[...]


def kernel(A, X, W1, b1, Wm, bm):
    n, d = X.shape
    h = W1.shape[1]
    k = max(1, int(0.5 * n))

    u = jax.random.uniform(jax.random.key(42), (n, 1), dtype=jnp.float32)
    g = -jnp.log(-jnp.log(u + 1e-08) + 1e-08)

    tr = 512
    n_tiles = n // tr
    c_tiles = 1
    u_tiles = n_tiles - c_tiles

    def a_map(i):
        return (jnp.where(i < n_tiles,
                          jnp.maximum(i - c_tiles, 0),
                          jnp.minimum(i - n_tiles, u_tiles - 1)), 0)

    def g_map(i):
        return (jnp.where(i < c_tiles, u_tiles + i,
                          jnp.clip(i - c_tiles, 0, n_tiles - 1)), 0)

    Y = pl.pallas_call(
        _y_kernel,
        out_shape=jax.ShapeDtypeStruct((n, h), jnp.bfloat16),
    )(X, W1)

    body = functools.partial(_fused_kernel, k=k, n=n, tr=tr,
                             n_tiles=n_tiles, c_tiles=c_tiles)
    A_aug = pl.pallas_call(
        body,
        grid=(2 * n_tiles,),
        in_specs=[
            pl.BlockSpec((tr, n), a_map),
            pl.BlockSpec((c_tiles * tr, n),
                         lambda i: (u_tiles // c_tiles, 0)),
            pl.BlockSpec((n, h), lambda i: (0, 0)),
            pl.BlockSpec((1, h), lambda i: (0, 0)),
            pl.BlockSpec((h, 1), lambda i: (0, 0)),
            pl.BlockSpec((1, 1), lambda i: (0, 0)),
            pl.BlockSpec((tr, 1), g_map),
        ],
        out_specs=pl.BlockSpec((tr, n),
                               lambda i: (jnp.maximum(i - n_tiles, 0), 0)),
        out_shape=jax.ShapeDtypeStruct((n, n), jnp.float32),
        scratch_shapes=[pltpu.VMEM((n, 1), jnp.float32),
                        pltpu.VMEM((1, n), jnp.float32),
                        pltpu.SMEM((2,), jnp.int32)],
        compiler_params=pltpu.CompilerParams(
            vmem_limit_bytes=60_000_000,
            dimension_semantics=("arbitrary",),
        ),
    )(A, A, Y, b1.reshape(1, h), Wm, bm.reshape(1, 1), g)
    return A_aug
